# baseline (speedup 1.0000x reference)
import jax
import jax.numpy as jnp
from jax import lax
from jax.experimental import pallas as pl
from jax.experimental.pallas import tpu as pltpu
from jax.experimental.pallas import tpu_sc as plsc

D_MODEL = 1024
D_FF = 4096
N_PRIM = 64
RANK = 64
TOP_K = 8
N_TOK = 4096

_NEG = -3.0e38
_BIG_I = 0x7FFFFFFF


def _topk_one(src, i_out, w_out, logits_v, idx_v, wgt_v):
  lane = lax.iota(jnp.int32, 16)
  pltpu.sync_copy(src, logits_v)
  vs = [logits_v[pl.ds(16 * i, 16)] for i in range(N_PRIM // 16)]
  ids = [lane + 16 * i for i in range(N_PRIM // 16)]
  sel_val = jnp.full((16,), _NEG, jnp.float32)
  sel_idx = jnp.zeros((16,), jnp.int32)
  for j in range(TOP_K):
    m = jnp.max(vs[0])
    for v in vs[1:]:
      m = jnp.maximum(m, jnp.max(v))
    t = jnp.int32(_BIG_I)
    for v, iv in zip(vs, ids):
      t = jnp.minimum(t, jnp.min(jnp.where(v == m, iv, _BIG_I)))
    sel_val = jnp.where(lane == j, m, sel_val)
    sel_idx = jnp.where(lane == j, t, sel_idx)
    vs = [jnp.where(iv == t, _NEG, v) for v, iv in zip(vs, ids)]
  m0 = jnp.max(sel_val)
  e = jnp.where(lane < TOP_K, jnp.exp(sel_val - m0), 0.0)
  s = jnp.sum(e)
  idx_v[...] = sel_idx
  wgt_v[...] = e / s
  pltpu.sync_copy(idx_v, i_out)
  pltpu.sync_copy(wgt_v, w_out)


def _sc_topk_body(l1_hbm, l2_hbm, i1_out, w1_out, i2_out, w2_out,
                  logits_v, idx_v, wgt_v):
  cid = lax.axis_index("c")
  sid = lax.axis_index("s")

  @pl.when(jnp.logical_and(cid == 0, sid == 0))
  def _():
    _topk_one(l1_hbm, i1_out, w1_out, logits_v, idx_v, wgt_v)

  @pl.when(jnp.logical_and(cid == 1, sid == 0))
  def _():
    _topk_one(l2_hbm, i2_out, w2_out, logits_v, idx_v, wgt_v)


def _sc_topk(l1, l2):
  f = pl.kernel(
      _sc_topk_body,
      out_type=(
          jax.ShapeDtypeStruct((16,), jnp.int32),
          jax.ShapeDtypeStruct((16,), jnp.float32),
          jax.ShapeDtypeStruct((16,), jnp.int32),
          jax.ShapeDtypeStruct((16,), jnp.float32),
      ),
      mesh=plsc.VectorSubcoreMesh(core_axis_name="c", subcore_axis_name="s"),
      compiler_params=pltpu.CompilerParams(needs_layout_passes=False),
      scratch_types=[
          pltpu.VMEM((N_PRIM,), jnp.float32),
          pltpu.VMEM((16,), jnp.int32),
          pltpu.VMEM((16,), jnp.float32),
      ],
  )
  return f(l1, l2)


ROWS = 1024
NT_STEPS = N_TOK // ROWS
KR = TOP_K * RANK
FCHUNK = 1024

_NT = (((1,), (1,)), ((), ()))

_GC1 = -2.0 * 0.7978845608028654
_GC3 = _GC1 * 0.044715


def _gelu_tanh(u):
  y = (_GC1 + _GC3 * (u * u)) * u
  return u / (1.0 + jnp.exp(y))


def _ffn_body(i1_ref, i2_ref, x_ref, a1t_hbm, b1_hbm, a2t_hbm, b2_hbm,
              w1_ref, w2_ref, out_ref,
              a1s, b1s, a2s, b2s, a1b, b1b, a2b, b2b, sem):
  t = pl.program_id(0)

  @pl.when(t == 0)
  def _prep():
    copies = []
    for k in range(TOP_K):
      i1k = i1_ref[k]
      i2k = i2_ref[k]
      copies.append(pltpu.make_async_copy(a1t_hbm.at[i1k], a1s.at[k], sem))
      copies.append(pltpu.make_async_copy(b1_hbm.at[i1k], b1s.at[k], sem))
      copies.append(pltpu.make_async_copy(a2t_hbm.at[i2k], a2s.at[k], sem))
      copies.append(pltpu.make_async_copy(b2_hbm.at[i2k], b2s.at[k], sem))
    for c in copies:
      c.start()
    for c in copies:
      c.wait()
    a1b[...] = jnp.concatenate(
        [a1s[k].astype(jnp.bfloat16) for k in range(TOP_K)], axis=0)
    b1b[...] = jnp.concatenate(
        [(b1s[k] * w1_ref[k]).astype(jnp.bfloat16) for k in range(TOP_K)],
        axis=0)
    a2b[...] = jnp.concatenate(
        [a2s[k].astype(jnp.bfloat16) for k in range(TOP_K)], axis=0)
    b2b[...] = jnp.concatenate(
        [(b2s[k] * w2_ref[k]).astype(jnp.bfloat16) for k in range(TOP_K)],
        axis=0)

  xb = x_ref[...].astype(jnp.bfloat16)
  xa = lax.dot_general(xb, a1b[...], _NT, preferred_element_type=jnp.float32)
  xa = xa.astype(jnp.bfloat16)
  ha = None
  for c in range(D_FF // FCHUNK):
    sl = pl.ds(c * FCHUNK, FCHUNK)
    u = jnp.dot(xa, b1b[:, sl], preferred_element_type=jnp.float32)
    g = _gelu_tanh(u).astype(jnp.bfloat16)
    p = lax.dot_general(g, a2b[:, sl], _NT,
                        preferred_element_type=jnp.float32)
    ha = p if ha is None else ha + p
  out_ref[...] = jnp.dot(ha.astype(jnp.bfloat16), b2b[...],
                         preferred_element_type=jnp.float32)


def _ffn(i1, i2, x, a1t, fc1_B, a2t, fc2_B, w1, w2):
  grid = (NT_STEPS,)
  return pl.pallas_call(
      _ffn_body,
      grid_spec=pltpu.PrefetchScalarGridSpec(
          num_scalar_prefetch=2,
          grid=grid,
          in_specs=[
              pl.BlockSpec((ROWS, D_MODEL), lambda t, i1, i2: (t, 0)),
              pl.BlockSpec(memory_space=pltpu.HBM),
              pl.BlockSpec(memory_space=pltpu.HBM),
              pl.BlockSpec(memory_space=pltpu.HBM),
              pl.BlockSpec(memory_space=pltpu.HBM),
              pl.BlockSpec(memory_space=pltpu.SMEM),
              pl.BlockSpec(memory_space=pltpu.SMEM),
          ],
          out_specs=pl.BlockSpec((ROWS, D_MODEL), lambda t, i1, i2: (t, 0)),
          scratch_shapes=[
              pltpu.VMEM((TOP_K, RANK, D_MODEL), jnp.float32),
              pltpu.VMEM((TOP_K, RANK, D_FF), jnp.float32),
              pltpu.VMEM((TOP_K, RANK, D_FF), jnp.float32),
              pltpu.VMEM((TOP_K, RANK, D_MODEL), jnp.float32),
              pltpu.VMEM((KR, D_MODEL), jnp.bfloat16),
              pltpu.VMEM((KR, D_FF), jnp.bfloat16),
              pltpu.VMEM((KR, D_FF), jnp.bfloat16),
              pltpu.VMEM((KR, D_MODEL), jnp.bfloat16),
              pltpu.SemaphoreType.DMA,
          ],
      ),
      out_shape=jax.ShapeDtypeStruct((N_TOK, D_MODEL), jnp.float32),
      compiler_params=pltpu.CompilerParams(
          dimension_semantics=("arbitrary",),
          vmem_limit_bytes=120 * 1024 * 1024,
      ),
  )(i1, i2, x, a1t, fc1_B, a2t, fc2_B, w1, w2)


def kernel(x, fc1_logits, fc1_A, fc1_B, fc2_logits, fc2_A, fc2_B):
  i1p, w1p, i2p, w2p = _sc_topk(fc1_logits, fc2_logits)
  a1t = jnp.transpose(fc1_A, (0, 2, 1))
  a2t = jnp.transpose(fc2_A, (0, 2, 1))
  return _ffn(i1p, i2p, x, a1t, fc1_B, a2t, fc2_B, w1p, w2p)


# speedup vs baseline: 1.0633x; 1.0633x over previous
import jax
import jax.numpy as jnp
from jax import lax
from jax.experimental import pallas as pl
from jax.experimental.pallas import tpu as pltpu
from jax.experimental.pallas import tpu_sc as plsc

D_MODEL = 1024
D_FF = 4096
N_PRIM = 64
RANK = 64
TOP_K = 8
N_TOK = 4096

_NEG = -3.0e38
_BIG_I = 0x7FFFFFFF


def _topk_one(src, i_out, w_out, logits_v, idx_v, wgt_v):
  lane = lax.iota(jnp.int32, 16)
  pltpu.sync_copy(src, logits_v)
  vs = [logits_v[pl.ds(16 * i, 16)] for i in range(N_PRIM // 16)]
  ids = [lane + 16 * i for i in range(N_PRIM // 16)]
  sel_val = jnp.full((16,), _NEG, jnp.float32)
  sel_idx = jnp.zeros((16,), jnp.int32)
  for j in range(TOP_K):
    m = jnp.max(vs[0])
    for v in vs[1:]:
      m = jnp.maximum(m, jnp.max(v))
    t = jnp.int32(_BIG_I)
    for v, iv in zip(vs, ids):
      t = jnp.minimum(t, jnp.min(jnp.where(v == m, iv, _BIG_I)))
    sel_val = jnp.where(lane == j, m, sel_val)
    sel_idx = jnp.where(lane == j, t, sel_idx)
    vs = [jnp.where(iv == t, _NEG, v) for v, iv in zip(vs, ids)]
  m0 = jnp.max(sel_val)
  e = jnp.where(lane < TOP_K, jnp.exp(sel_val - m0), 0.0)
  s = jnp.sum(e)
  idx_v[...] = sel_idx
  wgt_v[...] = e / s
  pltpu.sync_copy(idx_v, i_out)
  pltpu.sync_copy(wgt_v, w_out)


def _sc_topk_body(l1_hbm, l2_hbm, i1_out, w1_out, i2_out, w2_out,
                  logits_v, idx_v, wgt_v):
  cid = lax.axis_index("c")
  sid = lax.axis_index("s")

  @pl.when(jnp.logical_and(cid == 0, sid == 0))
  def _():
    _topk_one(l1_hbm, i1_out, w1_out, logits_v, idx_v, wgt_v)

  @pl.when(jnp.logical_and(cid == 1, sid == 0))
  def _():
    _topk_one(l2_hbm, i2_out, w2_out, logits_v, idx_v, wgt_v)


def _sc_topk(l1, l2):
  f = pl.kernel(
      _sc_topk_body,
      out_type=(
          jax.ShapeDtypeStruct((16,), jnp.int32),
          jax.ShapeDtypeStruct((16,), jnp.float32),
          jax.ShapeDtypeStruct((16,), jnp.int32),
          jax.ShapeDtypeStruct((16,), jnp.float32),
      ),
      mesh=plsc.VectorSubcoreMesh(core_axis_name="c", subcore_axis_name="s"),
      compiler_params=pltpu.CompilerParams(needs_layout_passes=False),
      scratch_types=[
          pltpu.VMEM((N_PRIM,), jnp.float32),
          pltpu.VMEM((16,), jnp.int32),
          pltpu.VMEM((16,), jnp.float32),
      ],
  )
  return f(l1, l2)


ROWS = 1024
NT_STEPS = N_TOK // ROWS
KR = TOP_K * RANK
FCHUNK = 1024

_NT = (((1,), (1,)), ((), ()))

_GC1 = -2.0 * 0.7978845608028654
_GC3 = _GC1 * 0.044715


def _gelu_tanh(u):
  y = (_GC1 + _GC3 * (u * u)) * u
  return u / (1.0 + jnp.exp(y))


def _ffn_body(i1_ref, i2_ref, x_ref, a1t_hbm, b1_hbm, a2t_hbm, b2_hbm,
              w1_ref, w2_ref, out_ref,
              a1s, b1s, a2s, b2s, a1b, b1b, a2b, b2b, sem):
  t = pl.program_id(0)

  @pl.when(t == 0)
  def _prep():
    copies = []
    for k in range(TOP_K):
      i1k = i1_ref[k]
      i2k = i2_ref[k]
      copies.append(pltpu.make_async_copy(a1t_hbm.at[i1k], a1s.at[k], sem))
      copies.append(pltpu.make_async_copy(b1_hbm.at[i1k], b1s.at[k], sem))
      copies.append(pltpu.make_async_copy(a2t_hbm.at[i2k], a2s.at[k], sem))
      copies.append(pltpu.make_async_copy(b2_hbm.at[i2k], b2s.at[k], sem))
    for c in copies:
      c.start()
    for c in copies:
      c.wait()
    a1b[...] = jnp.concatenate(
        [a1s[k].astype(jnp.bfloat16) for k in range(TOP_K)], axis=0)
    b1b[...] = jnp.concatenate(
        [(b1s[k] * w1_ref[k]).astype(jnp.bfloat16) for k in range(TOP_K)],
        axis=0)
    a2b[...] = jnp.concatenate(
        [a2s[k].astype(jnp.bfloat16) for k in range(TOP_K)], axis=0)
    b2b[...] = jnp.concatenate(
        [(b2s[k] * w2_ref[k]).astype(jnp.bfloat16) for k in range(TOP_K)],
        axis=0)

  xb = x_ref[...].astype(jnp.bfloat16)
  xa = lax.dot_general(xb, a1b[...], _NT, preferred_element_type=jnp.float32)
  xa = xa.astype(jnp.bfloat16)
  ha = None
  for c in range(D_FF // FCHUNK):
    sl = pl.ds(c * FCHUNK, FCHUNK)
    u = jnp.dot(xa, b1b[:, sl], preferred_element_type=jnp.float32)
    g = jax.nn.gelu(u).astype(jnp.bfloat16)
    p = lax.dot_general(g, a2b[:, sl], _NT,
                        preferred_element_type=jnp.float32)
    ha = p if ha is None else ha + p
  out_ref[...] = jnp.dot(ha.astype(jnp.bfloat16), b2b[...],
                         preferred_element_type=jnp.float32)


def _ffn(i1, i2, x, a1t, fc1_B, a2t, fc2_B, w1, w2):
  grid = (NT_STEPS,)
  return pl.pallas_call(
      _ffn_body,
      grid_spec=pltpu.PrefetchScalarGridSpec(
          num_scalar_prefetch=2,
          grid=grid,
          in_specs=[
              pl.BlockSpec((ROWS, D_MODEL), lambda t, i1, i2: (t, 0)),
              pl.BlockSpec(memory_space=pltpu.HBM),
              pl.BlockSpec(memory_space=pltpu.HBM),
              pl.BlockSpec(memory_space=pltpu.HBM),
              pl.BlockSpec(memory_space=pltpu.HBM),
              pl.BlockSpec(memory_space=pltpu.SMEM),
              pl.BlockSpec(memory_space=pltpu.SMEM),
          ],
          out_specs=pl.BlockSpec((ROWS, D_MODEL), lambda t, i1, i2: (t, 0)),
          scratch_shapes=[
              pltpu.VMEM((TOP_K, RANK, D_MODEL), jnp.float32),
              pltpu.VMEM((TOP_K, RANK, D_FF), jnp.float32),
              pltpu.VMEM((TOP_K, RANK, D_FF), jnp.float32),
              pltpu.VMEM((TOP_K, RANK, D_MODEL), jnp.float32),
              pltpu.VMEM((KR, D_MODEL), jnp.bfloat16),
              pltpu.VMEM((KR, D_FF), jnp.bfloat16),
              pltpu.VMEM((KR, D_FF), jnp.bfloat16),
              pltpu.VMEM((KR, D_MODEL), jnp.bfloat16),
              pltpu.SemaphoreType.DMA,
          ],
      ),
      out_shape=jax.ShapeDtypeStruct((N_TOK, D_MODEL), jnp.float32),
      compiler_params=pltpu.CompilerParams(
          dimension_semantics=("arbitrary",),
          vmem_limit_bytes=120 * 1024 * 1024,
      ),
  )(i1, i2, x, a1t, fc1_B, a2t, fc2_B, w1, w2)


def kernel(x, fc1_logits, fc1_A, fc1_B, fc2_logits, fc2_A, fc2_B):
  i1p, w1p, i2p, w2p = _sc_topk(fc1_logits, fc2_logits)
  a1t = jnp.transpose(fc1_A, (0, 2, 1))
  a2t = jnp.transpose(fc2_A, (0, 2, 1))
  return _ffn(i1p, i2p, x, a1t, fc1_B, a2t, fc2_B, w1p, w2p)


# speedup vs baseline: 1.0690x; 1.0054x over previous
import jax
import jax.numpy as jnp
from jax import lax
from jax.experimental import pallas as pl
from jax.experimental.pallas import tpu as pltpu
from jax.experimental.pallas import tpu_sc as plsc

D_MODEL = 1024
D_FF = 4096
N_PRIM = 64
RANK = 64
TOP_K = 8
N_TOK = 4096

_NEG = -3.0e38
_BIG_I = 0x7FFFFFFF


def _topk_one(src, i_out, w_out, logits_v, idx_v, wgt_v):
  lane = lax.iota(jnp.int32, 16)
  pltpu.sync_copy(src, logits_v)
  vs = [logits_v[pl.ds(16 * i, 16)] for i in range(N_PRIM // 16)]
  ids = [lane + 16 * i for i in range(N_PRIM // 16)]
  sel_val = jnp.full((16,), _NEG, jnp.float32)
  sel_idx = jnp.zeros((16,), jnp.int32)
  for j in range(TOP_K):
    m = jnp.max(vs[0])
    for v in vs[1:]:
      m = jnp.maximum(m, jnp.max(v))
    t = jnp.int32(_BIG_I)
    for v, iv in zip(vs, ids):
      t = jnp.minimum(t, jnp.min(jnp.where(v == m, iv, _BIG_I)))
    sel_val = jnp.where(lane == j, m, sel_val)
    sel_idx = jnp.where(lane == j, t, sel_idx)
    vs = [jnp.where(iv == t, _NEG, v) for v, iv in zip(vs, ids)]
  m0 = jnp.max(sel_val)
  e = jnp.where(lane < TOP_K, jnp.exp(sel_val - m0), 0.0)
  s = jnp.sum(e)
  idx_v[...] = sel_idx
  wgt_v[...] = e / s
  pltpu.sync_copy(idx_v, i_out)
  pltpu.sync_copy(wgt_v, w_out)


def _sc_topk_body(l1_hbm, l2_hbm, i1_out, w1_out, i2_out, w2_out,
                  logits_v, idx_v, wgt_v):
  cid = lax.axis_index("c")
  sid = lax.axis_index("s")

  @pl.when(jnp.logical_and(cid == 0, sid == 0))
  def _():
    _topk_one(l1_hbm, i1_out, w1_out, logits_v, idx_v, wgt_v)

  @pl.when(jnp.logical_and(cid == 1, sid == 0))
  def _():
    _topk_one(l2_hbm, i2_out, w2_out, logits_v, idx_v, wgt_v)


def _sc_topk(l1, l2):
  f = pl.kernel(
      _sc_topk_body,
      out_type=(
          jax.ShapeDtypeStruct((16,), jnp.int32),
          jax.ShapeDtypeStruct((16,), jnp.float32),
          jax.ShapeDtypeStruct((16,), jnp.int32),
          jax.ShapeDtypeStruct((16,), jnp.float32),
      ),
      mesh=plsc.VectorSubcoreMesh(core_axis_name="c", subcore_axis_name="s"),
      compiler_params=pltpu.CompilerParams(needs_layout_passes=False),
      scratch_types=[
          pltpu.VMEM((N_PRIM,), jnp.float32),
          pltpu.VMEM((16,), jnp.int32),
          pltpu.VMEM((16,), jnp.float32),
      ],
  )
  return f(l1, l2)


ROWS = 1024
NT_STEPS = N_TOK // ROWS
KR = TOP_K * RANK
FCHUNK = 2048

_NT = (((1,), (1,)), ((), ()))

_GC1 = -2.0 * 0.7978845608028654
_GC3 = _GC1 * 0.044715


def _gelu_tanh(u):
  y = (_GC1 + _GC3 * (u * u)) * u
  return u / (1.0 + jnp.exp(y))


def _ffn_body(i1_ref, i2_ref, x_ref, a1t_hbm, b1_hbm, a2t_hbm, b2_hbm,
              w1_ref, w2_ref, out_ref,
              a1s, b1s, a2s, b2s, a1b, b1b, a2b, b2b, sem):
  t = pl.program_id(0)

  @pl.when(t == 0)
  def _prep():
    copies = []
    for k in range(TOP_K):
      i1k = i1_ref[k]
      i2k = i2_ref[k]
      copies.append(pltpu.make_async_copy(a1t_hbm.at[i1k], a1s.at[k], sem))
      copies.append(pltpu.make_async_copy(b1_hbm.at[i1k], b1s.at[k], sem))
      copies.append(pltpu.make_async_copy(a2t_hbm.at[i2k], a2s.at[k], sem))
      copies.append(pltpu.make_async_copy(b2_hbm.at[i2k], b2s.at[k], sem))
    for c in copies:
      c.start()
    for c in copies:
      c.wait()
    a1b[...] = jnp.concatenate(
        [a1s[k].astype(jnp.bfloat16) for k in range(TOP_K)], axis=0)
    b1b[...] = jnp.concatenate(
        [(b1s[k] * w1_ref[k]).astype(jnp.bfloat16) for k in range(TOP_K)],
        axis=0)
    a2b[...] = jnp.concatenate(
        [a2s[k].astype(jnp.bfloat16) for k in range(TOP_K)], axis=0)
    b2b[...] = jnp.concatenate(
        [(b2s[k] * w2_ref[k]).astype(jnp.bfloat16) for k in range(TOP_K)],
        axis=0)

  xb = x_ref[...].astype(jnp.bfloat16)
  xa = lax.dot_general(xb, a1b[...], _NT, preferred_element_type=jnp.float32)
  xa = xa.astype(jnp.bfloat16)
  ha = None
  for c in range(D_FF // FCHUNK):
    sl = pl.ds(c * FCHUNK, FCHUNK)
    u = jnp.dot(xa, b1b[:, sl], preferred_element_type=jnp.float32)
    g = jax.nn.gelu(u).astype(jnp.bfloat16)
    p = lax.dot_general(g, a2b[:, sl], _NT,
                        preferred_element_type=jnp.float32)
    ha = p if ha is None else ha + p
  out_ref[...] = jnp.dot(ha.astype(jnp.bfloat16), b2b[...],
                         preferred_element_type=jnp.float32)


def _ffn(i1, i2, x, a1t, fc1_B, a2t, fc2_B, w1, w2):
  grid = (NT_STEPS,)
  return pl.pallas_call(
      _ffn_body,
      grid_spec=pltpu.PrefetchScalarGridSpec(
          num_scalar_prefetch=2,
          grid=grid,
          in_specs=[
              pl.BlockSpec((ROWS, D_MODEL), lambda t, i1, i2: (t, 0)),
              pl.BlockSpec(memory_space=pltpu.HBM),
              pl.BlockSpec(memory_space=pltpu.HBM),
              pl.BlockSpec(memory_space=pltpu.HBM),
              pl.BlockSpec(memory_space=pltpu.HBM),
              pl.BlockSpec(memory_space=pltpu.SMEM),
              pl.BlockSpec(memory_space=pltpu.SMEM),
          ],
          out_specs=pl.BlockSpec((ROWS, D_MODEL), lambda t, i1, i2: (t, 0)),
          scratch_shapes=[
              pltpu.VMEM((TOP_K, RANK, D_MODEL), jnp.float32),
              pltpu.VMEM((TOP_K, RANK, D_FF), jnp.float32),
              pltpu.VMEM((TOP_K, RANK, D_FF), jnp.float32),
              pltpu.VMEM((TOP_K, RANK, D_MODEL), jnp.float32),
              pltpu.VMEM((KR, D_MODEL), jnp.bfloat16),
              pltpu.VMEM((KR, D_FF), jnp.bfloat16),
              pltpu.VMEM((KR, D_FF), jnp.bfloat16),
              pltpu.VMEM((KR, D_MODEL), jnp.bfloat16),
              pltpu.SemaphoreType.DMA,
          ],
      ),
      out_shape=jax.ShapeDtypeStruct((N_TOK, D_MODEL), jnp.float32),
      compiler_params=pltpu.CompilerParams(
          dimension_semantics=("arbitrary",),
          vmem_limit_bytes=120 * 1024 * 1024,
      ),
  )(i1, i2, x, a1t, fc1_B, a2t, fc2_B, w1, w2)


def kernel(x, fc1_logits, fc1_A, fc1_B, fc2_logits, fc2_A, fc2_B):
  i1p, w1p, i2p, w2p = _sc_topk(fc1_logits, fc2_logits)
  a1t = jnp.transpose(fc1_A, (0, 2, 1))
  a2t = jnp.transpose(fc2_A, (0, 2, 1))
  return _ffn(i1p, i2p, x, a1t, fc1_B, a2t, fc2_B, w1p, w2p)
